# Initial kernel scaffold; baseline (speedup 1.0000x reference)
import jax
import jax.numpy as jnp
from jax.experimental import pallas as pl


def kernel(inputs, token_table, pos_table):
    raise NotImplementedError("write your pallas kernel here")


# speedup vs baseline: 1.9851x; 1.9851x over previous
import functools
import math

import jax
import jax.numpy as jnp
from jax import lax
from jax.experimental import pallas as pl
from jax.experimental.pallas import tpu as pltpu
from jax.experimental.pallas import tpu_sc as plsc


def _make_sc_kernel(total, V, D, S, C, rows_per_w, scale):
    chunks = rows_per_w // C
    mesh = plsc.VectorSubcoreMesh(core_axis_name="c", subcore_axis_name="s")

    @functools.partial(
        pl.kernel,
        mesh=mesh,
        compiler_params=pltpu.CompilerParams(use_tc_tiling_on_sc=False),
        out_type=jax.ShapeDtypeStruct((total, D), jnp.float32),
        scratch_types=[
            pltpu.VMEM((C,), jnp.int32),
            pltpu.VMEM((C, D), jnp.float32),
            pltpu.VMEM((S, D), jnp.float32),
            pltpu.SemaphoreType.DMA,
        ],
    )
    def sc_kernel(idx_hbm, table_hbm, pos_hbm, out_hbm, idx_v, rows_v, pos_v, sem):
        wid = lax.axis_index("s") * 2 + lax.axis_index("c")
        base = wid * rows_per_w
        pltpu.sync_copy(pos_hbm, pos_v)

        def chunk_body(ci, _):
            row0 = base + ci * C
            pltpu.sync_copy(idx_hbm.at[pl.ds(row0, C)], idx_v)
            pltpu.async_copy(table_hbm.at[idx_v], rows_v, sem).wait()

            def row_body(r, _):
                pr = lax.rem(r, S)
                for h in range(D // 16):
                    sl = pl.ds(h * 16, 16)
                    rows_v[r, sl] = rows_v[r, sl] * scale + pos_v[pr, sl]
                return 0

            lax.fori_loop(0, C, row_body, 0)
            pltpu.sync_copy(rows_v, out_hbm.at[pl.ds(row0, C)])
            return 0

        lax.fori_loop(0, chunks, chunk_body, 0)

    return sc_kernel


def kernel(inputs, token_table, pos_table):
    B, S = inputs.shape
    V, D = token_table.shape
    total = B * S
    NW = 32
    rows_per_w = total // NW
    C = 2 * S
    scale = float(math.sqrt(D))

    sc = _make_sc_kernel(total, V, D, S, C, rows_per_w, scale)
    idx_flat = inputs.reshape(total).astype(jnp.int32)
    out = sc(idx_flat, token_table, pos_table)
    return out.reshape(B, S, D)


# speedup vs baseline: 2.8201x; 1.4206x over previous
import functools
import math

import jax
import jax.numpy as jnp
from jax import lax
from jax.experimental import pallas as pl
from jax.experimental.pallas import tpu as pltpu
from jax.experimental.pallas import tpu_sc as plsc


def _make_sc_kernel(total, V, D, S, C, rows_per_w, scale):
    chunks = rows_per_w // C
    reps = C // S
    mesh = plsc.VectorSubcoreMesh(core_axis_name="c", subcore_axis_name="s")

    @functools.partial(
        pl.kernel,
        mesh=mesh,
        compiler_params=pltpu.CompilerParams(use_tc_tiling_on_sc=False),
        out_type=jax.ShapeDtypeStruct((total, D), jnp.float32),
        scratch_types=[
            pltpu.VMEM((C,), jnp.int32),
            pltpu.VMEM((C,), jnp.int32),
            pltpu.VMEM((C, D), jnp.float32),
            pltpu.VMEM((C, D), jnp.float32),
            pltpu.VMEM((S, D), jnp.float32),
            pltpu.SemaphoreType.DMA,
            pltpu.SemaphoreType.DMA,
        ],
    )
    def sc_kernel(idx_hbm, table_hbm, pos_hbm, out_hbm,
                  idx_v0, idx_v1, rows_v0, rows_v1, pos_v, gsem0, gsem1):
        idx_v = (idx_v0, idx_v1)
        rows_v = (rows_v0, rows_v1)
        gsem = (gsem0, gsem1)
        wid = lax.axis_index("s") * 2 + lax.axis_index("c")
        base = wid * rows_per_w
        pltpu.sync_copy(pos_hbm, pos_v)

        def issue_gather(ci, b):
            row0 = base + ci * C
            pltpu.sync_copy(idx_hbm.at[pl.ds(row0, C)], idx_v[b])
            pltpu.async_copy(table_hbm.at[idx_v[b]], rows_v[b], gsem[b])

        def wait_gather(b):
            pltpu.make_async_copy(table_hbm.at[idx_v[b]], rows_v[b], gsem[b]).wait()

        def compute(rv):
            def s_body(s, _):
                for h in range(D // 16):
                    sl = pl.ds(h * 16, 16)
                    p = pos_v[s, sl]
                    for k in range(reps):
                        r = s + k * S
                        rv[r, sl] = rv[r, sl] * scale + p
                return 0

            lax.fori_loop(0, S, s_body, 0)

        issue_gather(0, 0)

        def pair_body(i, _):
            for b in range(2):
                ci = i * 2 + b

                @pl.when(ci + 1 < chunks)
                def _():
                    issue_gather(ci + 1, 1 - b)

                wait_gather(b)
                compute(rows_v[b])
                pltpu.sync_copy(rows_v[b], out_hbm.at[pl.ds(base + ci * C, C)])
            return 0

        lax.fori_loop(0, chunks // 2, pair_body, 0)

    return sc_kernel


def kernel(inputs, token_table, pos_table):
    B, S = inputs.shape
    V, D = token_table.shape
    total = B * S
    NW = 32
    rows_per_w = total // NW
    C = 4 * S
    scale = float(math.sqrt(D))

    sc = _make_sc_kernel(total, V, D, S, C, rows_per_w, scale)
    idx_flat = inputs.reshape(total).astype(jnp.int32)
    out = sc(idx_flat, token_table, pos_table)
    return out.reshape(B, S, D)
